# baseline (speedup 1.0000x reference)
import functools

import jax
import jax.numpy as jnp
from jax import lax
from jax.experimental import pallas as pl
from jax.experimental.pallas import tpu as pltpu
from jax.experimental.pallas import tpu_sc as plsc

T = 16384
DIM_F = 32
DIM_CAT = 64
VOCAB = 100000
K_FLAGS = 32
MODEL_DIM = 256
TOTAL = 3 * DIM_F + 2 * DIM_CAT + DIM_F

_NC = 2
_NS = 16
_NW = _NC * _NS
_BPW = T // _NW

_BT = 2048


@functools.lru_cache(maxsize=1)
def _make_sc_gather():
    mesh = plsc.VectorSubcoreMesh(
        core_axis_name="c", subcore_axis_name="s",
        num_cores=_NC, num_subcores=_NS)

    ch = 128
    nch = _BPW // ch
    rpw = VOCAB // _NW

    bv = 2000
    nbv = VOCAB // 2 // bv

    def _widen_body(dx_hbm, pr_hbm, wdx_ref, wpr_ref,
                    sdt, sdb, spt, spb, sem):
        i = pl.program_id(0)
        cps = [
            pltpu.async_copy(dx_hbm.at[pl.ds(i * bv, bv)], sdt, sem),
            pltpu.async_copy(
                dx_hbm.at[pl.ds(VOCAB // 2 + i * bv, bv)], sdb, sem),
            pltpu.async_copy(pr_hbm.at[pl.ds(i * bv, bv)], spt, sem),
            pltpu.async_copy(
                pr_hbm.at[pl.ds(VOCAB // 2 + i * bv, bv)], spb, sem),
        ]
        for c in cps:
            c.wait()
        wdx_ref[...] = jnp.concatenate([sdt[...], sdb[...]], axis=1)
        wpr_ref[...] = jnp.concatenate([spt[...], spb[...]], axis=1)

    widen = pl.pallas_call(
        _widen_body,
        grid=(nbv,),
        in_specs=[
            pl.BlockSpec(memory_space=pl.ANY),
            pl.BlockSpec(memory_space=pl.ANY),
        ],
        out_specs=[
            pl.BlockSpec((bv, 2 * DIM_CAT), lambda i: (i, 0)),
            pl.BlockSpec((bv, 2 * DIM_CAT), lambda i: (i, 0)),
        ],
        out_shape=[
            jax.ShapeDtypeStruct((VOCAB // 2, 2 * DIM_CAT), jnp.float32),
            jax.ShapeDtypeStruct((VOCAB // 2, 2 * DIM_CAT), jnp.float32),
        ],
        scratch_shapes=[
            pltpu.VMEM((bv, DIM_CAT), jnp.float32),
            pltpu.VMEM((bv, DIM_CAT), jnp.float32),
            pltpu.VMEM((bv, DIM_CAT), jnp.float32),
            pltpu.VMEM((bv, DIM_CAT), jnp.float32),
            pltpu.SemaphoreType.DMA,
        ],
        compiler_params=pltpu.CompilerParams(
            dimension_semantics=("arbitrary",)),
    )

    @functools.partial(
        pl.kernel,
        mesh=mesh,
        compiler_params=pltpu.CompilerParams(needs_layout_passes=False),
        out_type=(jax.ShapeDtypeStruct((T, DIM_CAT), jnp.float32),
                  jax.ShapeDtypeStruct((T, DIM_CAT), jnp.float32)),
        scratch_types=[
            pltpu.VMEM((_BPW,), jnp.int32),
            pltpu.VMEM((_BPW,), jnp.int32),
            pltpu.VMEM((ch, 2 * DIM_CAT), jnp.float32),
            pltpu.VMEM((ch, 2 * DIM_CAT), jnp.float32),
            pltpu.VMEM((ch, DIM_CAT), jnp.float32),
            pltpu.SemaphoreType.DMA,
        ],
    )
    def gather2(dx_wide_hbm, proc_wide_hbm, dx_idx_hbm, proc_idx_hbm,
                dx_out, proc_out, idx_v, tid_v, buf_a, buf_b, sel_v, sem):
        wid = lax.axis_index("s") * _NC + lax.axis_index("c")
        base = wid * _BPW
        bufs = (buf_a, buf_b)
        lane = lax.iota(jnp.int32, 16)
        for tbl, idx_hbm, out in (
                (dx_wide_hbm, dx_idx_hbm, dx_out),
                (proc_wide_hbm, proc_idx_hbm, proc_out)):
            pltpu.sync_copy(idx_hbm.at[pl.ds(base, _BPW)], idx_v)
            for k in range(_BPW // 16):
                sl = pl.ds(k * 16, 16)
                v = idx_v[sl]
                half = (v >= VOCAB // 2).astype(jnp.int32)
                tid_v[sl] = v - half * (VOCAB // 2)
            cps = {}
            for c in range(min(2, nch)):
                cps[c] = pltpu.async_copy(
                    tbl.at[tid_v.at[pl.ds(c * ch, ch)]], bufs[c % 2], sem)
            for c in range(nch):
                cps[c].wait()
                buf = bufs[c % 2]

                def srow(i, _):
                    iv = plsc.load_gather(
                        idx_v, [lax.broadcast(c * ch + i, (16,))])
                    off16 = (iv >= VOCAB // 2).astype(jnp.int32) * DIM_CAT
                    il16 = lax.broadcast(i, (16,))
                    for q in range(DIM_CAT // 16):
                        vals = plsc.load_gather(
                            buf, [il16, off16 + q * 16 + lane])
                        plsc.store_scatter(
                            sel_v, [il16, q * 16 + lane], vals)
                    return 0
                lax.fori_loop(0, ch, srow, 0)
                pltpu.sync_copy(sel_v, out.at[pl.ds(base + c * ch, ch)])
                if c + 2 < nch:
                    cps[c + 2] = pltpu.async_copy(
                        tbl.at[tid_v.at[pl.ds((c + 2) * ch, ch)]],
                        bufs[c % 2], sem)

    def run(dx_emb, proc_emb, dx_idx, proc_idx):
        dx_wide, proc_wide = widen(dx_emb, proc_emb)
        return gather2(dx_wide, proc_wide, dx_idx, proc_idx)

    return run


def _tc_body(s_ref, meds_ref, dx_ref, pr_ref,
             wf_ref, bf_ref, ws_ref, bs_ref, me_ref,
             w1_ref, b1_ref, w2_ref, b2_ref, out_ref):
    f32 = jnp.float32
    h96 = lax.dot_general(s_ref[...], wf_ref[...],
                          (((0,), (0,)), ((), ())),
                          preferred_element_type=f32)
    h96 = jnp.maximum(h96 + bf_ref[...], 0.0)
    o96 = jnp.dot(h96, ws_ref[...], preferred_element_type=f32) + bs_ref[...]
    mask = (meds_ref[...] > 0.5).astype(f32)
    cnt = jnp.sum(mask, axis=1, keepdims=True)
    bag = jnp.dot(mask, me_ref[...], preferred_element_type=f32)
    o_meds = jnp.where(cnt > 0, bag / jnp.maximum(cnt, 1.0),
                       jnp.zeros_like(bag))
    concat = jnp.concatenate(
        [o96, dx_ref[...], pr_ref[...], o_meds], axis=1)
    h = jnp.maximum(
        jnp.dot(concat, w1_ref[...], preferred_element_type=f32)
        + b1_ref[...], 0.0)
    out_ref[...] = (jnp.dot(h, w2_ref[...], preferred_element_type=f32)
                    + b2_ref[...])


@functools.lru_cache(maxsize=1)
def _make_tc_dense():
    full = lambda shape: pl.BlockSpec(shape, lambda i: (0, 0))
    return pl.pallas_call(
        _tc_body,
        grid=(T // _BT,),
        in_specs=[
            pl.BlockSpec((8, _BT), lambda i: (0, i)),
            pl.BlockSpec((_BT, K_FLAGS), lambda i: (i, 0)),
            pl.BlockSpec((_BT, DIM_CAT), lambda i: (i, 0)),
            pl.BlockSpec((_BT, DIM_CAT), lambda i: (i, 0)),
            full((8, 3 * DIM_F)),
            full((1, 3 * DIM_F)),
            full((3 * DIM_F, 3 * DIM_F)),
            full((1, 3 * DIM_F)),
            full((K_FLAGS, DIM_F)),
            full((TOTAL, MODEL_DIM)),
            full((1, MODEL_DIM)),
            full((MODEL_DIM, MODEL_DIM)),
            full((1, MODEL_DIM)),
        ],
        out_specs=pl.BlockSpec((_BT, MODEL_DIM), lambda i: (i, 0)),
        out_shape=jax.ShapeDtypeStruct((T, MODEL_DIM), jnp.float32),
        compiler_params=pltpu.CompilerParams(
            dimension_semantics=("arbitrary",)),
    )


def kernel(log_dt, hr_value, hr_miss, sbp_value, sbp_miss, vent_value,
           vent_miss, dx_idx, proc_idx, meds_flags, hr_W1, hr_b1, hr_W2,
           hr_b2, sbp_W1, sbp_b1, sbp_W2, sbp_b2, vent_W1, vent_b1,
           vent_W2, vent_b2, dx_emb, proc_emb, meds_emb, proj_W1, proj_b1,
           proj_W2, proj_b2):
    f32 = jnp.float32
    dx_rows, pr_rows = _make_sc_gather()(
        dx_emb, proc_emb,
        dx_idx.astype(jnp.int32), proc_idx.astype(jnp.int32))

    s = jnp.stack([hr_value, hr_miss, sbp_value, sbp_miss,
                   vent_value, vent_miss, log_dt,
                   jnp.zeros_like(log_dt)], axis=0)
    z = jnp.zeros((1, DIM_F), f32)
    wf = jnp.concatenate([
        jnp.concatenate([hr_W1[0:1], hr_W1[1:2], z, z, z, z,
                         hr_W1[2:3], z], axis=0),
        jnp.concatenate([z, z, sbp_W1[0:1], sbp_W1[1:2], z, z,
                         sbp_W1[2:3], z], axis=0),
        jnp.concatenate([z, z, z, z, vent_W1[0:1], vent_W1[1:2],
                         vent_W1[2:3], z], axis=0),
    ], axis=1)
    bf = jnp.concatenate([hr_b1, sbp_b1, vent_b1])[None, :]
    zz = jnp.zeros((DIM_F, DIM_F), f32)
    ws = jnp.concatenate([
        jnp.concatenate([hr_W2, zz, zz], axis=1),
        jnp.concatenate([zz, sbp_W2, zz], axis=1),
        jnp.concatenate([zz, zz, vent_W2], axis=1),
    ], axis=0)
    bs = jnp.concatenate([hr_b2, sbp_b2, vent_b2])[None, :]

    return _make_tc_dense()(
        s, meds_flags, dx_rows, pr_rows,
        wf, bf, ws, bs, meds_emb,
        proj_W1, proj_b1[None, :], proj_W2, proj_b2[None, :])


# speedup vs baseline: 1.2232x; 1.2232x over previous
import functools

import jax
import jax.numpy as jnp
from jax import lax
from jax.experimental import pallas as pl
from jax.experimental.pallas import tpu as pltpu
from jax.experimental.pallas import tpu_sc as plsc

T = 16384
DIM_F = 32
DIM_CAT = 64
VOCAB = 100000
K_FLAGS = 32
MODEL_DIM = 256
TOTAL = 3 * DIM_F + 2 * DIM_CAT + DIM_F

_NC = 2
_NS = 16
_NW = _NC * _NS
_BPW = T // _NW

_BT = 2048


@functools.lru_cache(maxsize=1)
def _make_sc_gather():
    mesh = plsc.VectorSubcoreMesh(
        core_axis_name="c", subcore_axis_name="s",
        num_cores=_NC, num_subcores=_NS)

    ch = 128
    nch = _BPW // ch
    rpw = VOCAB // _NW

    bv = 2000
    nbv = VOCAB // 2 // bv

    def _widen_body(dx_hbm, pr_hbm, wdx_ref, wpr_ref,
                    sdt, sdb, spt, spb, sem):
        i = pl.program_id(0)
        cps = [
            pltpu.async_copy(dx_hbm.at[pl.ds(i * bv, bv)], sdt, sem),
            pltpu.async_copy(
                dx_hbm.at[pl.ds(VOCAB // 2 + i * bv, bv)], sdb, sem),
            pltpu.async_copy(pr_hbm.at[pl.ds(i * bv, bv)], spt, sem),
            pltpu.async_copy(
                pr_hbm.at[pl.ds(VOCAB // 2 + i * bv, bv)], spb, sem),
        ]
        for c in cps:
            c.wait()
        wdx_ref[...] = jnp.concatenate([sdt[...], sdb[...]], axis=1)
        wpr_ref[...] = jnp.concatenate([spt[...], spb[...]], axis=1)

    widen = pl.pallas_call(
        _widen_body,
        grid=(nbv,),
        in_specs=[
            pl.BlockSpec(memory_space=pl.ANY),
            pl.BlockSpec(memory_space=pl.ANY),
        ],
        out_specs=[
            pl.BlockSpec((bv, 2 * DIM_CAT), lambda i: (i, 0)),
            pl.BlockSpec((bv, 2 * DIM_CAT), lambda i: (i, 0)),
        ],
        out_shape=[
            jax.ShapeDtypeStruct((VOCAB // 2, 2 * DIM_CAT), jnp.float32),
            jax.ShapeDtypeStruct((VOCAB // 2, 2 * DIM_CAT), jnp.float32),
        ],
        scratch_shapes=[
            pltpu.VMEM((bv, DIM_CAT), jnp.float32),
            pltpu.VMEM((bv, DIM_CAT), jnp.float32),
            pltpu.VMEM((bv, DIM_CAT), jnp.float32),
            pltpu.VMEM((bv, DIM_CAT), jnp.float32),
            pltpu.SemaphoreType.DMA,
        ],
        compiler_params=pltpu.CompilerParams(
            dimension_semantics=("arbitrary",)),
    )

    @functools.partial(
        pl.kernel,
        mesh=mesh,
        compiler_params=pltpu.CompilerParams(needs_layout_passes=False),
        out_type=(jax.ShapeDtypeStruct((T, DIM_CAT), jnp.float32),
                  jax.ShapeDtypeStruct((T, DIM_CAT), jnp.float32)),
        scratch_types=[
            pltpu.VMEM((_BPW,), jnp.int32),
            pltpu.VMEM((_BPW,), jnp.int32),
            pltpu.VMEM((ch, 2 * DIM_CAT), jnp.float32),
            pltpu.VMEM((ch, 2 * DIM_CAT), jnp.float32),
            pltpu.VMEM((ch, DIM_CAT), jnp.float32),
            pltpu.SemaphoreType.DMA,
        ],
    )
    def gather2(dx_wide_hbm, proc_wide_hbm, dx_idx_hbm, proc_idx_hbm,
                dx_out, proc_out, idx_v, tid_v, buf_a, buf_b, sel_v, sem):
        wid = lax.axis_index("s") * _NC + lax.axis_index("c")
        base = wid * _BPW
        bufs = (buf_a, buf_b)
        lane = lax.iota(jnp.int32, 16)
        for tbl, idx_hbm, out in (
                (dx_wide_hbm, dx_idx_hbm, dx_out),
                (proc_wide_hbm, proc_idx_hbm, proc_out)):
            pltpu.sync_copy(idx_hbm.at[pl.ds(base, _BPW)], idx_v)
            for k in range(_BPW // 16):
                sl = pl.ds(k * 16, 16)
                tid_v[sl] = lax.shift_right_logical(idx_v[sl], 1)
            cps = {}
            for c in range(min(2, nch)):
                cps[c] = pltpu.async_copy(
                    tbl.at[tid_v.at[pl.ds(c * ch, ch)]], bufs[c % 2], sem)
            for c in range(nch):
                cps[c].wait()
                buf = bufs[c % 2]

                def srow(i, _):
                    iv = plsc.load_gather(
                        idx_v, [lax.broadcast(c * ch + i, (16,))])
                    off16 = (iv & 1) * DIM_CAT
                    il16 = lax.broadcast(i, (16,))
                    for q in range(DIM_CAT // 16):
                        vals = plsc.load_gather(
                            buf, [il16, off16 + q * 16 + lane])
                        plsc.store_scatter(
                            sel_v, [il16, q * 16 + lane], vals)
                    return 0
                lax.fori_loop(0, ch, srow, 0)
                pltpu.sync_copy(sel_v, out.at[pl.ds(base + c * ch, ch)])
                if c + 2 < nch:
                    cps[c + 2] = pltpu.async_copy(
                        tbl.at[tid_v.at[pl.ds((c + 2) * ch, ch)]],
                        bufs[c % 2], sem)

    def run(dx_emb, proc_emb, dx_idx, proc_idx):
        dx_wide = jnp.reshape(dx_emb, (VOCAB // 2, 2 * DIM_CAT))
        proc_wide = jnp.reshape(proc_emb, (VOCAB // 2, 2 * DIM_CAT))
        return gather2(dx_wide, proc_wide, dx_idx, proc_idx)

    return run


def _tc_body(s_ref, meds_ref, dx_ref, pr_ref,
             wf_ref, bf_ref, ws_ref, bs_ref, me_ref,
             w1_ref, b1_ref, w2_ref, b2_ref, out_ref):
    f32 = jnp.float32
    h96 = lax.dot_general(s_ref[...], wf_ref[...],
                          (((0,), (0,)), ((), ())),
                          preferred_element_type=f32)
    h96 = jnp.maximum(h96 + bf_ref[...], 0.0)
    o96 = jnp.dot(h96, ws_ref[...], preferred_element_type=f32) + bs_ref[...]
    mask = (meds_ref[...] > 0.5).astype(f32)
    cnt = jnp.sum(mask, axis=1, keepdims=True)
    bag = jnp.dot(mask, me_ref[...], preferred_element_type=f32)
    o_meds = jnp.where(cnt > 0, bag / jnp.maximum(cnt, 1.0),
                       jnp.zeros_like(bag))
    concat = jnp.concatenate(
        [o96, dx_ref[...], pr_ref[...], o_meds], axis=1)
    h = jnp.maximum(
        jnp.dot(concat, w1_ref[...], preferred_element_type=f32)
        + b1_ref[...], 0.0)
    out_ref[...] = (jnp.dot(h, w2_ref[...], preferred_element_type=f32)
                    + b2_ref[...])


@functools.lru_cache(maxsize=1)
def _make_tc_dense():
    full = lambda shape: pl.BlockSpec(shape, lambda i: (0, 0))
    return pl.pallas_call(
        _tc_body,
        grid=(T // _BT,),
        in_specs=[
            pl.BlockSpec((8, _BT), lambda i: (0, i)),
            pl.BlockSpec((_BT, K_FLAGS), lambda i: (i, 0)),
            pl.BlockSpec((_BT, DIM_CAT), lambda i: (i, 0)),
            pl.BlockSpec((_BT, DIM_CAT), lambda i: (i, 0)),
            full((8, 3 * DIM_F)),
            full((1, 3 * DIM_F)),
            full((3 * DIM_F, 3 * DIM_F)),
            full((1, 3 * DIM_F)),
            full((K_FLAGS, DIM_F)),
            full((TOTAL, MODEL_DIM)),
            full((1, MODEL_DIM)),
            full((MODEL_DIM, MODEL_DIM)),
            full((1, MODEL_DIM)),
        ],
        out_specs=pl.BlockSpec((_BT, MODEL_DIM), lambda i: (i, 0)),
        out_shape=jax.ShapeDtypeStruct((T, MODEL_DIM), jnp.float32),
        compiler_params=pltpu.CompilerParams(
            dimension_semantics=("arbitrary",)),
    )


def kernel(log_dt, hr_value, hr_miss, sbp_value, sbp_miss, vent_value,
           vent_miss, dx_idx, proc_idx, meds_flags, hr_W1, hr_b1, hr_W2,
           hr_b2, sbp_W1, sbp_b1, sbp_W2, sbp_b2, vent_W1, vent_b1,
           vent_W2, vent_b2, dx_emb, proc_emb, meds_emb, proj_W1, proj_b1,
           proj_W2, proj_b2):
    f32 = jnp.float32
    dx_rows, pr_rows = _make_sc_gather()(
        dx_emb, proc_emb,
        dx_idx.astype(jnp.int32), proc_idx.astype(jnp.int32))

    s = jnp.stack([hr_value, hr_miss, sbp_value, sbp_miss,
                   vent_value, vent_miss, log_dt,
                   jnp.zeros_like(log_dt)], axis=0)
    z = jnp.zeros((1, DIM_F), f32)
    wf = jnp.concatenate([
        jnp.concatenate([hr_W1[0:1], hr_W1[1:2], z, z, z, z,
                         hr_W1[2:3], z], axis=0),
        jnp.concatenate([z, z, sbp_W1[0:1], sbp_W1[1:2], z, z,
                         sbp_W1[2:3], z], axis=0),
        jnp.concatenate([z, z, z, z, vent_W1[0:1], vent_W1[1:2],
                         vent_W1[2:3], z], axis=0),
    ], axis=1)
    bf = jnp.concatenate([hr_b1, sbp_b1, vent_b1])[None, :]
    zz = jnp.zeros((DIM_F, DIM_F), f32)
    ws = jnp.concatenate([
        jnp.concatenate([hr_W2, zz, zz], axis=1),
        jnp.concatenate([zz, sbp_W2, zz], axis=1),
        jnp.concatenate([zz, zz, vent_W2], axis=1),
    ], axis=0)
    bs = jnp.concatenate([hr_b2, sbp_b2, vent_b2])[None, :]

    return _make_tc_dense()(
        s, meds_flags, dx_rows, pr_rows,
        wf, bf, ws, bs, meds_emb,
        proj_W1, proj_b1[None, :], proj_W2, proj_b2[None, :])


# speedup vs baseline: 1.2941x; 1.0579x over previous
import functools

import jax
import jax.numpy as jnp
from jax import lax
from jax.experimental import pallas as pl
from jax.experimental.pallas import tpu as pltpu
from jax.experimental.pallas import tpu_sc as plsc

T = 16384
DIM_F = 32
DIM_CAT = 64
VOCAB = 100000
K_FLAGS = 32
MODEL_DIM = 256
TOTAL = 3 * DIM_F + 2 * DIM_CAT + DIM_F

_NC = 2
_NS = 16
_NW = _NC * _NS
_BPW = T // _NW
_SPLIT = 51200

_BT = 2048


@functools.lru_cache(maxsize=1)
def _make_sc_gather():
    mesh = plsc.VectorSubcoreMesh(
        core_axis_name="c", subcore_axis_name="s",
        num_cores=_NC, num_subcores=_NS)

    ch = 128
    nch = _BPW // ch

    @functools.partial(
        pl.kernel,
        mesh=mesh,
        compiler_params=pltpu.CompilerParams(use_tc_tiling_on_sc=False),
        out_type=(jax.ShapeDtypeStruct((T, DIM_CAT), jnp.float32),
                  jax.ShapeDtypeStruct((T, DIM_CAT), jnp.float32)),
        scratch_types=[
            pltpu.VMEM((_BPW,), jnp.int32),
            pltpu.VMEM((_BPW,), jnp.int32),
            pltpu.VMEM((_BPW, DIM_CAT), jnp.float32),
            pltpu.VMEM((_BPW, DIM_CAT), jnp.float32),
            pltpu.SemaphoreType.DMA,
        ],
    )
    def gather2(dx_emb_hbm, proc_emb_hbm, dx_idx_hbm, proc_idx_hbm,
                dx_out, proc_out, idx_dx, idx_pr, rows_dx, rows_pr, sem):
        wid = lax.axis_index("s") * _NC + lax.axis_index("c")
        base = wid * _BPW
        pltpu.sync_copy(dx_idx_hbm.at[pl.ds(base, _BPW)], idx_dx)
        pltpu.sync_copy(proc_idx_hbm.at[pl.ds(base, _BPW)], idx_pr)
        copies = []
        for j in range(nch):
            sl = pl.ds(j * ch, ch)
            copies.append(pltpu.async_copy(
                dx_emb_hbm.at[idx_dx.at[sl]], rows_dx.at[sl], sem))
            copies.append(pltpu.async_copy(
                proc_emb_hbm.at[idx_pr.at[sl]], rows_pr.at[sl], sem))
        for c in copies:
            c.wait()
        pltpu.sync_copy(rows_dx, dx_out.at[pl.ds(base, _BPW)])
        pltpu.sync_copy(rows_pr, proc_out.at[pl.ds(base, _BPW)])

    return gather2


def _tc_body(s_ref, meds_ref, dx_ref, pr_ref,
             wf_ref, bf_ref, ws_ref, bs_ref, me_ref,
             w1_ref, b1_ref, w2_ref, b2_ref, out_ref):
    f32 = jnp.float32
    h96 = lax.dot_general(s_ref[...], wf_ref[...],
                          (((0,), (0,)), ((), ())),
                          preferred_element_type=f32)
    h96 = jnp.maximum(h96 + bf_ref[...], 0.0)
    o96 = jnp.dot(h96, ws_ref[...], preferred_element_type=f32) + bs_ref[...]
    mask = (meds_ref[...] > 0.5).astype(f32)
    cnt = jnp.sum(mask, axis=1, keepdims=True)
    bag = jnp.dot(mask, me_ref[...], preferred_element_type=f32)
    o_meds = jnp.where(cnt > 0, bag / jnp.maximum(cnt, 1.0),
                       jnp.zeros_like(bag))
    concat = jnp.concatenate(
        [o96, dx_ref[...], pr_ref[...], o_meds], axis=1)
    h = jnp.maximum(
        jnp.dot(concat, w1_ref[...], preferred_element_type=f32)
        + b1_ref[...], 0.0)
    out_ref[...] = (jnp.dot(h, w2_ref[...], preferred_element_type=f32)
                    + b2_ref[...])


@functools.lru_cache(maxsize=1)
def _make_tc_dense():
    full = lambda shape: pl.BlockSpec(shape, lambda i: (0, 0))
    return pl.pallas_call(
        _tc_body,
        grid=(T // _BT,),
        in_specs=[
            pl.BlockSpec((8, _BT), lambda i: (0, i)),
            pl.BlockSpec((_BT, K_FLAGS), lambda i: (i, 0)),
            pl.BlockSpec((_BT, DIM_CAT), lambda i: (i, 0)),
            pl.BlockSpec((_BT, DIM_CAT), lambda i: (i, 0)),
            full((8, 3 * DIM_F)),
            full((1, 3 * DIM_F)),
            full((3 * DIM_F, 3 * DIM_F)),
            full((1, 3 * DIM_F)),
            full((K_FLAGS, DIM_F)),
            full((TOTAL, MODEL_DIM)),
            full((1, MODEL_DIM)),
            full((MODEL_DIM, MODEL_DIM)),
            full((1, MODEL_DIM)),
        ],
        out_specs=pl.BlockSpec((_BT, MODEL_DIM), lambda i: (i, 0)),
        out_shape=jax.ShapeDtypeStruct((T, MODEL_DIM), jnp.float32),
        compiler_params=pltpu.CompilerParams(
            dimension_semantics=("arbitrary",)),
    )


def kernel(log_dt, hr_value, hr_miss, sbp_value, sbp_miss, vent_value,
           vent_miss, dx_idx, proc_idx, meds_flags, hr_W1, hr_b1, hr_W2,
           hr_b2, sbp_W1, sbp_b1, sbp_W2, sbp_b2, vent_W1, vent_b1,
           vent_W2, vent_b2, dx_emb, proc_emb, meds_emb, proj_W1, proj_b1,
           proj_W2, proj_b2):
    f32 = jnp.float32
    dx_rows, pr_rows = _make_sc_gather()(
        dx_emb, proc_emb,
        dx_idx.astype(jnp.int32), proc_idx.astype(jnp.int32))

    s = jnp.stack([hr_value, hr_miss, sbp_value, sbp_miss,
                   vent_value, vent_miss, log_dt,
                   jnp.zeros_like(log_dt)], axis=0)
    z = jnp.zeros((1, DIM_F), f32)
    wf = jnp.concatenate([
        jnp.concatenate([hr_W1[0:1], hr_W1[1:2], z, z, z, z,
                         hr_W1[2:3], z], axis=0),
        jnp.concatenate([z, z, sbp_W1[0:1], sbp_W1[1:2], z, z,
                         sbp_W1[2:3], z], axis=0),
        jnp.concatenate([z, z, z, z, vent_W1[0:1], vent_W1[1:2],
                         vent_W1[2:3], z], axis=0),
    ], axis=1)
    bf = jnp.concatenate([hr_b1, sbp_b1, vent_b1])[None, :]
    zz = jnp.zeros((DIM_F, DIM_F), f32)
    ws = jnp.concatenate([
        jnp.concatenate([hr_W2, zz, zz], axis=1),
        jnp.concatenate([zz, sbp_W2, zz], axis=1),
        jnp.concatenate([zz, zz, vent_W2], axis=1),
    ], axis=0)
    bs = jnp.concatenate([hr_b2, sbp_b2, vent_b2])[None, :]

    return _make_tc_dense()(
        s, meds_flags, dx_rows, pr_rows,
        wf, bf, ws, bs, meds_emb,
        proj_W1, proj_b1[None, :], proj_W2, proj_b2[None, :])


# speedup vs baseline: 1.3004x; 1.0049x over previous
import functools

import jax
import jax.numpy as jnp
from jax import lax
from jax.experimental import pallas as pl
from jax.experimental.pallas import tpu as pltpu
from jax.experimental.pallas import tpu_sc as plsc

T = 16384
DIM_F = 32
DIM_CAT = 64
VOCAB = 100000
K_FLAGS = 32
MODEL_DIM = 256
TOTAL = 3 * DIM_F + 2 * DIM_CAT + DIM_F

_NC = 2
_NS = 16
_NW = _NC * _NS
_BPW = T // _NW
_SPLIT = 51200

_BT = 2048


@functools.lru_cache(maxsize=1)
def _make_sc_gather():
    mesh = plsc.VectorSubcoreMesh(
        core_axis_name="c", subcore_axis_name="s",
        num_cores=_NC, num_subcores=_NS)

    ch = 128
    nch = _BPW // ch

    @functools.partial(
        pl.kernel,
        mesh=mesh,
        compiler_params=pltpu.CompilerParams(use_tc_tiling_on_sc=False),
        out_type=(jax.ShapeDtypeStruct((T, DIM_CAT), jnp.float32),
                  jax.ShapeDtypeStruct((T, DIM_CAT), jnp.float32)),
        scratch_types=[
            pltpu.VMEM((_BPW,), jnp.int32),
            pltpu.VMEM((_BPW,), jnp.int32),
            pltpu.VMEM((_BPW, DIM_CAT), jnp.float32),
            pltpu.VMEM((_BPW, DIM_CAT), jnp.float32),
            pltpu.SemaphoreType.DMA,
        ],
    )
    def gather2(dx_emb_hbm, proc_emb_hbm, dx_idx_hbm, proc_idx_hbm,
                dx_out, proc_out, idx_dx, idx_pr, rows_dx, rows_pr, sem):
        wid = lax.axis_index("s") * _NC + lax.axis_index("c")
        base = wid * _BPW
        pltpu.sync_copy(dx_idx_hbm.at[pl.ds(base, _BPW)], idx_dx)
        pltpu.sync_copy(proc_idx_hbm.at[pl.ds(base, _BPW)], idx_pr)
        copies = []
        for j in range(nch):
            sl = pl.ds(j * ch, ch)
            copies.append(pltpu.async_copy(
                dx_emb_hbm.at[idx_dx.at[sl]], rows_dx.at[sl], sem))
            copies.append(pltpu.async_copy(
                proc_emb_hbm.at[idx_pr.at[sl]], rows_pr.at[sl], sem))
        for c in copies:
            c.wait()
        pltpu.sync_copy(rows_dx, dx_out.at[pl.ds(base, _BPW)])
        pltpu.sync_copy(rows_pr, proc_out.at[pl.ds(base, _BPW)])

    return gather2


def _tc_body(hv_ref, hm_ref, sv_ref, sm_ref, vv_ref, vm_ref, ld_ref,
             meds_ref, dx_ref, pr_ref,
             wf_ref, bf_ref, ws_ref, bs_ref, me_ref,
             w1_ref, b1_ref, w2_ref, b2_ref, out_ref):
    f32 = jnp.float32
    s = jnp.stack([hv_ref[...], hm_ref[...], sv_ref[...], sm_ref[...],
                   vv_ref[...], vm_ref[...], ld_ref[...]], axis=0)
    h96 = lax.dot_general(s, wf_ref[...],
                          (((0,), (0,)), ((), ())),
                          preferred_element_type=f32)
    h96 = jnp.maximum(h96 + bf_ref[...], 0.0)
    o96 = jnp.dot(h96, ws_ref[...], preferred_element_type=f32) + bs_ref[...]
    mask = (meds_ref[...] > 0.5).astype(f32)
    cnt = jnp.sum(mask, axis=1, keepdims=True)
    bag = jnp.dot(mask, me_ref[...], preferred_element_type=f32)
    o_meds = jnp.where(cnt > 0, bag / jnp.maximum(cnt, 1.0),
                       jnp.zeros_like(bag))
    concat = jnp.concatenate(
        [o96, dx_ref[...], pr_ref[...], o_meds], axis=1)
    h = jnp.maximum(
        jnp.dot(concat, w1_ref[...], preferred_element_type=f32)
        + b1_ref[...], 0.0)
    out_ref[...] = (jnp.dot(h, w2_ref[...], preferred_element_type=f32)
                    + b2_ref[...])


@functools.lru_cache(maxsize=1)
def _make_tc_dense():
    full = lambda shape: pl.BlockSpec(shape, lambda i: (0, 0))
    return pl.pallas_call(
        _tc_body,
        grid=(T // _BT,),
        in_specs=[
            pl.BlockSpec((_BT,), lambda i: (i,)),
            pl.BlockSpec((_BT,), lambda i: (i,)),
            pl.BlockSpec((_BT,), lambda i: (i,)),
            pl.BlockSpec((_BT,), lambda i: (i,)),
            pl.BlockSpec((_BT,), lambda i: (i,)),
            pl.BlockSpec((_BT,), lambda i: (i,)),
            pl.BlockSpec((_BT,), lambda i: (i,)),
            pl.BlockSpec((_BT, K_FLAGS), lambda i: (i, 0)),
            pl.BlockSpec((_BT, DIM_CAT), lambda i: (i, 0)),
            pl.BlockSpec((_BT, DIM_CAT), lambda i: (i, 0)),
            full((7, 3 * DIM_F)),
            full((1, 3 * DIM_F)),
            full((3 * DIM_F, 3 * DIM_F)),
            full((1, 3 * DIM_F)),
            full((K_FLAGS, DIM_F)),
            full((TOTAL, MODEL_DIM)),
            full((1, MODEL_DIM)),
            full((MODEL_DIM, MODEL_DIM)),
            full((1, MODEL_DIM)),
        ],
        out_specs=pl.BlockSpec((_BT, MODEL_DIM), lambda i: (i, 0)),
        out_shape=jax.ShapeDtypeStruct((T, MODEL_DIM), jnp.float32),
        compiler_params=pltpu.CompilerParams(
            dimension_semantics=("arbitrary",)),
    )


def kernel(log_dt, hr_value, hr_miss, sbp_value, sbp_miss, vent_value,
           vent_miss, dx_idx, proc_idx, meds_flags, hr_W1, hr_b1, hr_W2,
           hr_b2, sbp_W1, sbp_b1, sbp_W2, sbp_b2, vent_W1, vent_b1,
           vent_W2, vent_b2, dx_emb, proc_emb, meds_emb, proj_W1, proj_b1,
           proj_W2, proj_b2):
    f32 = jnp.float32
    dx_rows, pr_rows = _make_sc_gather()(
        dx_emb, proc_emb,
        dx_idx.astype(jnp.int32), proc_idx.astype(jnp.int32))

    z = jnp.zeros((1, DIM_F), f32)
    wf = jnp.concatenate([
        jnp.concatenate([hr_W1[0:1], hr_W1[1:2], z, z, z, z,
                         hr_W1[2:3]], axis=0),
        jnp.concatenate([z, z, sbp_W1[0:1], sbp_W1[1:2], z, z,
                         sbp_W1[2:3]], axis=0),
        jnp.concatenate([z, z, z, z, vent_W1[0:1], vent_W1[1:2],
                         vent_W1[2:3]], axis=0),
    ], axis=1)
    bf = jnp.concatenate([hr_b1, sbp_b1, vent_b1])[None, :]
    zz = jnp.zeros((DIM_F, DIM_F), f32)
    ws = jnp.concatenate([
        jnp.concatenate([hr_W2, zz, zz], axis=1),
        jnp.concatenate([zz, sbp_W2, zz], axis=1),
        jnp.concatenate([zz, zz, vent_W2], axis=1),
    ], axis=0)
    bs = jnp.concatenate([hr_b2, sbp_b2, vent_b2])[None, :]

    return _make_tc_dense()(
        hr_value, hr_miss, sbp_value, sbp_miss, vent_value, vent_miss,
        log_dt, meds_flags, dx_rows, pr_rows,
        wf, bf, ws, bs, meds_emb,
        proj_W1, proj_b1[None, :], proj_W2, proj_b2[None, :])


# speedup vs baseline: 1.3012x; 1.0006x over previous
import functools

import jax
import jax.numpy as jnp
from jax import lax
from jax.experimental import pallas as pl
from jax.experimental.pallas import tpu as pltpu
from jax.experimental.pallas import tpu_sc as plsc

T = 16384
DIM_F = 32
DIM_CAT = 64
VOCAB = 100000
K_FLAGS = 32
MODEL_DIM = 256
TOTAL = 3 * DIM_F + 2 * DIM_CAT + DIM_F

_NC = 2
_NS = 16
_NW = _NC * _NS
_BPW = T // _NW

_BT = 2048


@functools.lru_cache(maxsize=1)
def _make_sc_gather():
    mesh = plsc.VectorSubcoreMesh(
        core_axis_name="c", subcore_axis_name="s",
        num_cores=_NC, num_subcores=_NS)

    ch = 128
    nch = _BPW // ch

    @functools.partial(
        pl.kernel,
        mesh=mesh,
        compiler_params=pltpu.CompilerParams(use_tc_tiling_on_sc=False),
        out_type=(jax.ShapeDtypeStruct((T, DIM_CAT), jnp.float32),
                  jax.ShapeDtypeStruct((T, DIM_CAT), jnp.float32)),
        scratch_types=[
            pltpu.VMEM((_BPW,), jnp.int32),
            pltpu.VMEM((_BPW,), jnp.int32),
            pltpu.VMEM((_BPW, DIM_CAT), jnp.float32),
            pltpu.VMEM((_BPW, DIM_CAT), jnp.float32),
            pltpu.SemaphoreType.DMA,
        ],
    )
    def gather2(dx_emb_hbm, proc_emb_hbm, dx_idx_hbm, proc_idx_hbm,
                dx_out, proc_out, idx_dx, idx_pr, rows_dx, rows_pr, sem):
        wid = lax.axis_index("s") * _NC + lax.axis_index("c")
        base = wid * _BPW
        pltpu.sync_copy(dx_idx_hbm.at[pl.ds(base, _BPW)], idx_dx)
        pltpu.sync_copy(proc_idx_hbm.at[pl.ds(base, _BPW)], idx_pr)
        copies = []
        for j in range(nch):
            sl = pl.ds(j * ch, ch)
            copies.append(pltpu.async_copy(
                dx_emb_hbm.at[idx_dx.at[sl]], rows_dx.at[sl], sem))
            copies.append(pltpu.async_copy(
                proc_emb_hbm.at[idx_pr.at[sl]], rows_pr.at[sl], sem))
        for c in copies:
            c.wait()
        pltpu.sync_copy(rows_dx, dx_out.at[pl.ds(base, _BPW)])
        pltpu.sync_copy(rows_pr, proc_out.at[pl.ds(base, _BPW)])

    return gather2


def _tc_body(hv_ref, hm_ref, sv_ref, sm_ref, vv_ref, vm_ref, ld_ref,
             meds_ref, dx_ref, pr_ref,
             wf_ref, bf_ref, ws_ref, bs_ref, me_ref,
             w1_ref, b1_ref, w2_ref, b2_ref, out_ref):
    f32 = jnp.float32
    s = jnp.stack([hv_ref[...], hm_ref[...], sv_ref[...], sm_ref[...],
                   vv_ref[...], vm_ref[...], ld_ref[...]], axis=0)
    h96 = lax.dot_general(s, wf_ref[...],
                          (((0,), (0,)), ((), ())),
                          preferred_element_type=f32)
    h96 = jnp.maximum(h96 + bf_ref[...], 0.0)
    o96 = jnp.dot(h96, ws_ref[...], preferred_element_type=f32) + bs_ref[...]
    mask = (meds_ref[...] > 0.5).astype(f32)
    cnt = jnp.sum(mask, axis=1, keepdims=True)
    bag = jnp.dot(mask, me_ref[...], preferred_element_type=f32)
    o_meds = jnp.where(cnt > 0, bag / jnp.maximum(cnt, 1.0),
                       jnp.zeros_like(bag))
    concat = jnp.concatenate(
        [o96, dx_ref[...], pr_ref[...], o_meds], axis=1)
    h = jnp.maximum(
        jnp.dot(concat, w1_ref[...], preferred_element_type=f32)
        + b1_ref[...], 0.0)
    out_ref[...] = (jnp.dot(h, w2_ref[...], preferred_element_type=f32)
                    + b2_ref[...])


@functools.lru_cache(maxsize=1)
def _make_tc_dense():
    full = lambda shape: pl.BlockSpec(shape, lambda i: (0, 0))
    return pl.pallas_call(
        _tc_body,
        grid=(T // _BT,),
        in_specs=[
            pl.BlockSpec((_BT,), lambda i: (i,)),
            pl.BlockSpec((_BT,), lambda i: (i,)),
            pl.BlockSpec((_BT,), lambda i: (i,)),
            pl.BlockSpec((_BT,), lambda i: (i,)),
            pl.BlockSpec((_BT,), lambda i: (i,)),
            pl.BlockSpec((_BT,), lambda i: (i,)),
            pl.BlockSpec((_BT,), lambda i: (i,)),
            pl.BlockSpec((_BT, K_FLAGS), lambda i: (i, 0)),
            pl.BlockSpec((_BT, DIM_CAT), lambda i: (i, 0)),
            pl.BlockSpec((_BT, DIM_CAT), lambda i: (i, 0)),
            full((7, 3 * DIM_F)),
            full((1, 3 * DIM_F)),
            full((3 * DIM_F, 3 * DIM_F)),
            full((1, 3 * DIM_F)),
            full((K_FLAGS, DIM_F)),
            full((TOTAL, MODEL_DIM)),
            full((1, MODEL_DIM)),
            full((MODEL_DIM, MODEL_DIM)),
            full((1, MODEL_DIM)),
        ],
        out_specs=pl.BlockSpec((_BT, MODEL_DIM), lambda i: (i, 0)),
        out_shape=jax.ShapeDtypeStruct((T, MODEL_DIM), jnp.float32),
        compiler_params=pltpu.CompilerParams(
            dimension_semantics=("arbitrary",)),
    )


def kernel(log_dt, hr_value, hr_miss, sbp_value, sbp_miss, vent_value,
           vent_miss, dx_idx, proc_idx, meds_flags, hr_W1, hr_b1, hr_W2,
           hr_b2, sbp_W1, sbp_b1, sbp_W2, sbp_b2, vent_W1, vent_b1,
           vent_W2, vent_b2, dx_emb, proc_emb, meds_emb, proj_W1, proj_b1,
           proj_W2, proj_b2):
    f32 = jnp.float32
    dx_rows, pr_rows = _make_sc_gather()(
        dx_emb, proc_emb,
        dx_idx.astype(jnp.int32), proc_idx.astype(jnp.int32))

    z = jnp.zeros((1, DIM_F), f32)
    wf = jnp.concatenate([
        jnp.concatenate([hr_W1[0:1], hr_W1[1:2], z, z, z, z,
                         hr_W1[2:3]], axis=0),
        jnp.concatenate([z, z, sbp_W1[0:1], sbp_W1[1:2], z, z,
                         sbp_W1[2:3]], axis=0),
        jnp.concatenate([z, z, z, z, vent_W1[0:1], vent_W1[1:2],
                         vent_W1[2:3]], axis=0),
    ], axis=1)
    bf = jnp.concatenate([hr_b1, sbp_b1, vent_b1])[None, :]
    zz = jnp.zeros((DIM_F, DIM_F), f32)
    ws = jnp.concatenate([
        jnp.concatenate([hr_W2, zz, zz], axis=1),
        jnp.concatenate([zz, sbp_W2, zz], axis=1),
        jnp.concatenate([zz, zz, vent_W2], axis=1),
    ], axis=0)
    bs = jnp.concatenate([hr_b2, sbp_b2, vent_b2])[None, :]

    return _make_tc_dense()(
        hr_value, hr_miss, sbp_value, sbp_miss, vent_value, vent_miss,
        log_dt, meds_flags, dx_rows, pr_rows,
        wf, bf, ws, bs, meds_emb,
        proj_W1, proj_b1[None, :], proj_W2, proj_b2[None, :])
